# baseline (speedup 1.0000x reference)
import functools

import jax
import jax.numpy as jnp
from jax import lax
from jax.experimental import pallas as pl
from jax.experimental.pallas import tpu as pltpu
from jax.experimental.pallas import tpu_sc as plsc

_N = 4096
_E = 131072
_H = 32
_T = 4
_NC = 2
_NS = 16
_NW = _NC * _NS
_NPT = _N // _NW
_CAP = 8192
_CHB = 16384
_GCH = 256
_L = 16

_mesh = plsc.VectorSubcoreMesh(core_axis_name="c", subcore_axis_name="s")
_params = pltpu.CompilerParams(needs_layout_passes=False)


def _wid():
    return lax.axis_index("s") * _NC + lax.axis_index("c")


_SUB = _CAP // _L
_STR = _CHB // _L
_GF = 128


@functools.partial(
    pl.kernel,
    mesh=_mesh,
    compiler_params=_params,
    out_type=[
        jax.ShapeDtypeStruct((_NW * _CAP,), jnp.int32),
        jax.ShapeDtypeStruct((_NW * _CAP,), jnp.int32),
        jax.ShapeDtypeStruct((_NW * _CAP,), jnp.float32),
        jax.ShapeDtypeStruct((_NW * _L,), jnp.int32),
    ],
    scratch_types=[
        pltpu.VMEM((_CHB,), jnp.int32),
        pltpu.VMEM((_CHB,), jnp.int32),
        pltpu.VMEM((_CAP,), jnp.int32),
        pltpu.VMEM((_CAP,), jnp.int32),
        pltpu.VMEM((_CAP,), jnp.int32),
        pltpu.VMEM((_CAP,), jnp.int32),
        pltpu.VMEM((_CAP,), jnp.float32),
        pltpu.VMEM((_CAP,), jnp.int32),
        pltpu.VMEM((_GCH, 128), jnp.float32),
        pltpu.VMEM((_L,), jnp.int32),
        pltpu.SemaphoreType.DMA,
    ],
)
def _prep(src_hbm, dst_hbm, emat_hbm, src_l, dst_l, feat_l, cnt_hbm,
          sbuf, dbuf, src_r, dst_r, src_o, dst_o, feat_o, rowb,
          grows0, cstage, sem0):
    w = _wid()
    iota = lax.broadcasted_iota(jnp.int32, (_L,), 0)
    zeros16 = jnp.zeros((_L,), jnp.int32)
    lane_base = iota * _SUB
    stride_idx = iota * _STR

    dump_dst = zeros16 + (w * _NPT + _NPT)

    def prefill(i, _):
        src_r[pl.ds(i * _L, _L)] = zeros16
        dst_r[pl.ds(i * _L, _L)] = dump_dst
        src_o[pl.ds(i * _L, _L)] = zeros16
        dst_o[pl.ds(i * _L, _L)] = dump_dst
        return 0

    lax.fori_loop(0, _CAP // _L, prefill, 0)

    def chunk_body(c, poscnt):
        pltpu.sync_copy(src_hbm.at[pl.ds(c * _CHB, _CHB)], sbuf)
        pltpu.sync_copy(dst_hbm.at[pl.ds(c * _CHB, _CHB)], dbuf)

        def vec_body(i, poscnt):
            s_v = sbuf[pl.ds(i * _L, _L)]
            d_v = dbuf[pl.ds(i * _L, _L)]
            m = (d_v >> 7) == w
            mi = jnp.where(m, jnp.int32(1), jnp.int32(0))
            pos = lane_base + jnp.minimum(poscnt, _SUB - 1)
            plsc.store_scatter(src_r, [pos], s_v, mask=m)
            plsc.store_scatter(dst_r, [pos], d_v, mask=m)
            return poscnt + mi

        return lax.fori_loop(0, _CHB // _L, vec_body, poscnt)

    poscnt = lax.fori_loop(0, _E // _CHB, chunk_body, zeros16)

    d_r = jnp.int32(0)
    for r in range(_L):
        c_r = poscnt[r]
        nv_r = (c_r + _L - 1) // _L

        def cp(k, _, r=r, d_r=d_r):
            v_s = src_r[pl.ds(r * _SUB + k * _L, _L)]
            v_d = dst_r[pl.ds(r * _SUB + k * _L, _L)]
            src_o[pl.ds(d_r + k * _L, _L)] = v_s
            dst_o[pl.ds(d_r + k * _L, _L)] = v_d
            return 0

        lax.fori_loop(0, nv_r, cp, 0)
        d_r = d_r + nv_r * _L

    cntp = ((d_r + _GCH - 1) // _GCH) * _GCH

    def lin_body(i, _):
        s_v = src_o[pl.ds(i * _L, _L)]
        d_v = dst_o[pl.ds(i * _L, _L)]
        rowb[pl.ds(i * _L, _L)] = (s_v * _N + d_v) >> 7
        return 0

    lax.fori_loop(0, cntp // _L, lin_body, 0)

    def gch_body(c, _):
        pltpu.async_copy(
            emat_hbm.at[rowb.at[pl.ds(c * _GCH, _GCH)]], grows0, sem0
        ).wait()

        def ext_body(i, _):
            s_v = src_o[pl.ds(c * _GCH + i * _L, _L)]
            d_v = dst_o[pl.ds(c * _GCH + i * _L, _L)]
            col = (s_v * _N + d_v) & 127
            feat_o[pl.ds(c * _GCH + i * _L, _L)] = plsc.load_gather(
                grows0, [i * _L + iota, col])
            return 0

        lax.fori_loop(0, _GCH // _L, ext_body, 0)
        return 0

    lax.fori_loop(0, cntp // _GCH, gch_body, 0)

    pltpu.sync_copy(src_o, src_l.at[pl.ds(w * _CAP, _CAP)])
    pltpu.sync_copy(dst_o, dst_l.at[pl.ds(w * _CAP, _CAP)])
    pltpu.sync_copy(feat_o, feat_l.at[pl.ds(w * _CAP, _CAP)])
    cstage[pl.ds(0, _L)] = zeros16 + cntp
    pltpu.sync_copy(cstage, cnt_hbm.at[pl.ds(w * _L, _L)])


@functools.partial(
    pl.kernel,
    mesh=_mesh,
    compiler_params=_params,
    out_type=jax.ShapeDtypeStruct((2 * _N * 16,), jnp.float32),
    scratch_types=[
        pltpu.VMEM((_N * 16,), jnp.float32),
        pltpu.VMEM((_CAP,), jnp.int32),
        pltpu.VMEM((_CAP,), jnp.int32),
        pltpu.VMEM((_CAP,), jnp.float32),
        pltpu.VMEM((_L,), jnp.int32),
        pltpu.VMEM(((_NPT + 1) * 16 * _L,), jnp.float32),
        pltpu.VMEM((_H,), jnp.float32),
    ],
)
def _msgmax(a0_hbm, a1_hbm, src_l, dst_l, feat_l, cnt_hbm, wf_hbm, u_hbm,
            ap, srcb, dstb, featb, cstage, u_loc, wfb):
    w = _wid()
    base = w * _NPT
    pltpu.sync_copy(src_l.at[pl.ds(w * _CAP, _CAP)], srcb)
    pltpu.sync_copy(dst_l.at[pl.ds(w * _CAP, _CAP)], dstb)
    pltpu.sync_copy(feat_l.at[pl.ds(w * _CAP, _CAP)], featb)
    pltpu.sync_copy(cnt_hbm.at[pl.ds(w * _L, _L)], cstage)
    pltpu.sync_copy(wf_hbm, wfb)
    n_p = cstage[pl.ds(0, _L)][0]
    neg = jnp.full((_L,), -jnp.inf, jnp.float32)

    for h, a_hbm in ((0, a0_hbm), (1, a1_hbm)):
        pltpu.sync_copy(a_hbm, ap)
        wfh = wfb[pl.ds(h * _L, _L)]

        def fill(i, _):
            u_loc[pl.ds(i * _L, _L)] = neg
            return 0

        lax.fori_loop(0, (_NPT + 1) * _L, fill, 0)

        cpysz = (_NPT + 1) * 16

        def g_body(g, _):
            s_v = srcb[pl.ds(g * _L, _L)]
            d_v = dstb[pl.ds(g * _L, _L)]
            f_v = featb[pl.ds(g * _L, _L)]
            for j in range(_L):
                s = s_v[j]
                f = f_v[j]
                off = j * cpysz + (d_v[j] - base) * 16
                v = ap[pl.ds(s * _L, _L)] + f * wfh
                u_loc[pl.ds(off, _L)] = jnp.maximum(u_loc[pl.ds(off, _L)], v)
            return 0

        lax.fori_loop(0, n_p // _L, g_body, 0)

        def merge(r, _):
            acc = u_loc[pl.ds(r * _L, _L)]
            for j in range(1, _L):
                acc = jnp.maximum(acc, u_loc[pl.ds(j * cpysz + r * _L, _L)])
            u_loc[pl.ds(r * _L, _L)] = acc
            return 0

        lax.fori_loop(0, _NPT, merge, 0)
        pltpu.sync_copy(
            u_loc.at[pl.ds(0, _NPT * 16)],
            u_hbm.at[pl.ds(h * _N * 16 + base * 16, _NPT * 16)])


def _encode_body(x_ref, h_ref, p_ref, we_ref, be_ref,
                 wms_ref, z_ref, a_ref):
    inp = jnp.concatenate([x_ref[...], h_ref[...], p_ref[...]], axis=1)
    z = jnp.dot(inp, we_ref[...],
                preferred_element_type=jnp.float32) + be_ref[...]
    z_ref[...] = z
    a_ref[...] = jnp.dot(
        z.astype(jnp.bfloat16), wms_ref[...],
        preferred_element_type=jnp.float32)


_encode = pl.pallas_call(
    _encode_body,
    out_shape=[jax.ShapeDtypeStruct((_N, _H), jnp.float32),
               jax.ShapeDtypeStruct((_N, _H), jnp.float32)],
)


def _update_body(z_ref, u_ref, wmd_ref, bm_ref, wu_ref, bu_ref,
                 wt_ref, bt_ref, wnn_ref, bnn_ref,
                 wup_ref, bup_ref,
                 nh_ref, ns_ref, nne_ref, stop_ref):
    z = z_ref[...]
    zq = z.astype(jnp.bfloat16)
    u_raw = u_ref[...]
    b = jnp.dot(zq, wmd_ref[...], preferred_element_type=jnp.float32)
    u = jnp.where(jnp.isfinite(u_raw), u_raw + b + bm_ref[...], 0.0)
    nh = jnp.dot(jnp.concatenate([z, u], axis=1), wu_ref[...],
                 preferred_element_type=jnp.float32) + bu_ref[...]
    nh_ref[...] = nh
    loc = jnp.dot(nh, wt_ref[...], preferred_element_type=jnp.float32)
    mloc = jnp.dot(jnp.mean(nh, axis=0, keepdims=True), wt_ref[...],
                   preferred_element_type=jnp.float32)
    stop_ref[...] = jax.nn.sigmoid(
        jnp.maximum(jnp.max(loc, axis=0, keepdims=True), mloc) + bt_ref[...])
    nne = jnp.dot(jnp.concatenate([nh, z], axis=1), wnn_ref[...],
                  preferred_element_type=jnp.float32) + bnn_ref[...]
    nne_ref[...] = nne
    ns_ref[...] = jnp.dot(jnp.concatenate([nh, z, nne], axis=1), wup_ref[...],
                          preferred_element_type=jnp.float32) + bup_ref[...]


_update = pl.pallas_call(
    _update_body,
    out_shape=[jax.ShapeDtypeStruct((_N, _H), jnp.float32),
               jax.ShapeDtypeStruct((_N, 1), jnp.float32),
               jax.ShapeDtypeStruct((_N, 1), jnp.float32),
               jax.ShapeDtypeStruct((1, 1), jnp.float32)],
)

_SIG = list(range(0, _H, 2)) + list(range(1, _H, 2))


def kernel(states, priority, edges_mat, edge_index,
           W_enc, b_enc, W_M, b_M, W_U, b_U,
           W_nn, b_nn, W_up, b_up, W_t, b_t):
    src = edge_index[0]
    dst = edge_index[1]
    emat128 = edges_mat.astype(jnp.bfloat16).astype(
        jnp.float32).reshape(_N * _N // 128, 128)
    src_l, dst_l, feat_l, cnt = _prep(src, dst, emat128)

    bf = jnp.bfloat16
    be = b_enc[None]
    wms = W_M[:32].astype(bf)
    wmd = W_M[32:64].astype(bf)
    wf = W_M[64].astype(bf).astype(jnp.float32)
    bm = b_M[None]
    bu = b_U[None]
    bnn = b_nn[None]
    bup = b_up[None]
    bt = b_t[None]

    hidden = jnp.zeros((_N, _H), jnp.float32)
    x = states[0][:, None]
    prio = priority[:, None]
    pred_all = [x]
    pred_stop = [jnp.zeros((1, 1), jnp.float32)]
    pred_next = []
    for _ in range(_T - 1):
        z, a_f = _encode(x, hidden, prio, W_enc, be, wms)
        a0 = a_f[:, :16].reshape(_N * 16)
        a1 = a_f[:, 16:].reshape(_N * 16)
        uflat = _msgmax(a0, a1, src_l, dst_l, feat_l, cnt, wf)
        u_raw = jnp.concatenate(
            [uflat[:_N * 16].reshape(_N, 16),
             uflat[_N * 16:].reshape(_N, 16)], axis=1)
        nh, ns, nne, stop = _update(z, u_raw, wmd, bm, W_U, bu,
                                    W_t, bt, W_nn, bnn, W_up, bup)
        hidden = nh
        x = ns
        pred_all.append(ns)
        pred_stop.append(stop)
        pred_next.append(nne)

    preds = jnp.stack(pred_all, axis=1).reshape(_T, _N)
    preds_stop = jnp.stack(pred_stop, axis=1)
    preds_nextnode = jnp.stack(pred_next, axis=1)
    return (preds, preds_stop, preds_nextnode)


# speedup vs baseline: 1.0993x; 1.0993x over previous
import functools

import jax
import jax.numpy as jnp
from jax import lax
from jax.experimental import pallas as pl
from jax.experimental.pallas import tpu as pltpu
from jax.experimental.pallas import tpu_sc as plsc

_N = 4096
_E = 131072
_H = 32
_T = 4
_NC = 2
_NS = 16
_NW = _NC * _NS
_NPT = _N // _NW
_CAP = 8192
_CHB = 16384
_GCH = 256
_L = 16

_mesh = plsc.VectorSubcoreMesh(core_axis_name="c", subcore_axis_name="s")
_params = pltpu.CompilerParams(needs_layout_passes=False)


def _wid():
    return lax.axis_index("s") * _NC + lax.axis_index("c")


_SUB = _CAP // _L
_STR = _CHB // _L
_GF = 128


@functools.partial(
    pl.kernel,
    mesh=_mesh,
    compiler_params=_params,
    out_type=[
        jax.ShapeDtypeStruct((_NW * _CAP,), jnp.int32),
        jax.ShapeDtypeStruct((_NW * _CAP,), jnp.int32),
        jax.ShapeDtypeStruct((_NW * _CAP,), jnp.float32),
        jax.ShapeDtypeStruct((_NW * _L,), jnp.int32),
    ],
    scratch_types=[
        pltpu.VMEM((_CHB,), jnp.int32),
        pltpu.VMEM((_CHB,), jnp.int32),
        pltpu.VMEM((_CAP,), jnp.int32),
        pltpu.VMEM((_CAP,), jnp.int32),
        pltpu.VMEM((_CAP,), jnp.int32),
        pltpu.VMEM((_CAP,), jnp.int32),
        pltpu.VMEM((_CAP,), jnp.float32),
        pltpu.VMEM((_CAP,), jnp.int32),
        pltpu.VMEM((_GCH, 128), jnp.float32),
        pltpu.VMEM((_L,), jnp.int32),
        pltpu.SemaphoreType.DMA,
    ],
)
def _prep(src_hbm, dst_hbm, emat_hbm, src_l, dst_l, feat_l, cnt_hbm,
          sbuf, dbuf, src_r, dst_r, src_o, dst_o, feat_o, rowb,
          grows0, cstage, sem0):
    w = _wid()
    iota = lax.broadcasted_iota(jnp.int32, (_L,), 0)
    zeros16 = jnp.zeros((_L,), jnp.int32)
    lane_base = iota * _SUB
    stride_idx = iota * _STR

    dump_dst = zeros16 + (w * _NPT + _NPT)

    def prefill(i, _):
        src_r[pl.ds(i * _L, _L)] = zeros16
        dst_r[pl.ds(i * _L, _L)] = dump_dst
        src_o[pl.ds(i * _L, _L)] = zeros16
        dst_o[pl.ds(i * _L, _L)] = dump_dst
        return 0

    lax.fori_loop(0, _CAP // _L, prefill, 0)

    def chunk_body(c, poscnt):
        pltpu.sync_copy(src_hbm.at[pl.ds(c * _CHB, _CHB)], sbuf)
        pltpu.sync_copy(dst_hbm.at[pl.ds(c * _CHB, _CHB)], dbuf)

        def vec_body(i, poscnt):
            s_v = sbuf[pl.ds(i * _L, _L)]
            d_v = dbuf[pl.ds(i * _L, _L)]
            m = (d_v >> 7) == w
            mi = jnp.where(m, jnp.int32(1), jnp.int32(0))
            pos = lane_base + jnp.minimum(poscnt, _SUB - 1)
            plsc.store_scatter(src_r, [pos], s_v, mask=m)
            plsc.store_scatter(dst_r, [pos], d_v, mask=m)
            return poscnt + mi

        return lax.fori_loop(0, _CHB // _L, vec_body, poscnt)

    poscnt = lax.fori_loop(0, _E // _CHB, chunk_body, zeros16)

    d_r = jnp.int32(0)
    for r in range(_L):
        c_r = poscnt[r]
        nv_r = (c_r + _L - 1) // _L

        def cp(k, _, r=r, d_r=d_r):
            v_s = src_r[pl.ds(r * _SUB + k * _L, _L)]
            v_d = dst_r[pl.ds(r * _SUB + k * _L, _L)]
            src_o[pl.ds(d_r + k * _L, _L)] = v_s
            dst_o[pl.ds(d_r + k * _L, _L)] = v_d
            return 0

        lax.fori_loop(0, nv_r, cp, 0)
        d_r = d_r + nv_r * _L

    cntp = ((d_r + _GCH - 1) // _GCH) * _GCH

    def lin_body(i, _):
        s_v = src_o[pl.ds(i * _L, _L)]
        d_v = dst_o[pl.ds(i * _L, _L)]
        rowb[pl.ds(i * _L, _L)] = (s_v * _N + d_v) >> 7
        return 0

    lax.fori_loop(0, cntp // _L, lin_body, 0)

    def gch_body(c, _):
        pltpu.async_copy(
            emat_hbm.at[rowb.at[pl.ds(c * _GCH, _GCH)]], grows0, sem0
        ).wait()

        def ext_body(i, _):
            s_v = src_o[pl.ds(c * _GCH + i * _L, _L)]
            d_v = dst_o[pl.ds(c * _GCH + i * _L, _L)]
            col = (s_v * _N + d_v) & 127
            feat_o[pl.ds(c * _GCH + i * _L, _L)] = plsc.load_gather(
                grows0, [i * _L + iota, col])
            return 0

        lax.fori_loop(0, _GCH // _L, ext_body, 0)
        return 0

    lax.fori_loop(0, cntp // _GCH, gch_body, 0)

    pltpu.sync_copy(src_o, src_l.at[pl.ds(w * _CAP, _CAP)])
    pltpu.sync_copy(dst_o, dst_l.at[pl.ds(w * _CAP, _CAP)])
    pltpu.sync_copy(feat_o, feat_l.at[pl.ds(w * _CAP, _CAP)])
    cstage[pl.ds(0, _L)] = zeros16 + cntp
    pltpu.sync_copy(cstage, cnt_hbm.at[pl.ds(w * _L, _L)])


@functools.partial(
    pl.kernel,
    mesh=_mesh,
    compiler_params=_params,
    out_type=jax.ShapeDtypeStruct((2 * _N * 16,), jnp.float32),
    scratch_types=[
        pltpu.VMEM((_N * 16,), jnp.float32),
        pltpu.VMEM((_CAP,), jnp.int32),
        pltpu.VMEM((_CAP,), jnp.int32),
        pltpu.VMEM((_CAP,), jnp.float32),
        pltpu.VMEM((_L,), jnp.int32),
        pltpu.VMEM(((_NPT + 1) * 16,), jnp.float32),
        pltpu.VMEM((_H,), jnp.float32),
    ],
)
def _msgmax(a0_hbm, a1_hbm, src_l, dst_l, feat_l, cnt_hbm, wf_hbm, u_hbm,
            ap, srcb, dstb, featb, cstage, u_loc, wfb):
    w = _wid()
    base = w * _NPT
    pltpu.sync_copy(src_l.at[pl.ds(w * _CAP, _CAP)], srcb)
    pltpu.sync_copy(dst_l.at[pl.ds(w * _CAP, _CAP)], dstb)
    pltpu.sync_copy(feat_l.at[pl.ds(w * _CAP, _CAP)], featb)
    pltpu.sync_copy(cnt_hbm.at[pl.ds(w * _L, _L)], cstage)
    pltpu.sync_copy(wf_hbm, wfb)
    n_p = cstage[pl.ds(0, _L)][0]
    neg = jnp.full((_L,), -jnp.inf, jnp.float32)

    for h, a_hbm in ((0, a0_hbm), (1, a1_hbm)):
        pltpu.sync_copy(a_hbm, ap)
        wfh = wfb[pl.ds(h * _L, _L)]

        def fill(i, _):
            u_loc[pl.ds(i * _L, _L)] = neg
            return 0

        lax.fori_loop(0, _NPT + 1, fill, 0)

        def g_body(g, _):
            s_v = srcb[pl.ds(g * _L, _L)]
            d_v = dstb[pl.ds(g * _L, _L)]
            f_v = featb[pl.ds(g * _L, _L)]
            for j in range(_L):
                s = s_v[j]
                f = f_v[j]
                off = (d_v[j] - base) * 16
                v = ap[pl.ds(s * _L, _L)] + f * wfh
                u_loc[pl.ds(off, _L)] = jnp.maximum(u_loc[pl.ds(off, _L)], v)
            return 0

        lax.fori_loop(0, n_p // _L, g_body, 0)
        pltpu.sync_copy(
            u_loc.at[pl.ds(0, _NPT * 16)],
            u_hbm.at[pl.ds(h * _N * 16 + base * 16, _NPT * 16)])


def _encode_body(x_ref, h_ref, p_ref, we_ref, be_ref,
                 wms_ref, z_ref, a_ref):
    inp = jnp.concatenate([x_ref[...], h_ref[...], p_ref[...]], axis=1)
    z = jnp.dot(inp, we_ref[...],
                preferred_element_type=jnp.float32) + be_ref[...]
    z_ref[...] = z
    a_ref[...] = jnp.dot(
        z.astype(jnp.bfloat16), wms_ref[...],
        preferred_element_type=jnp.float32)


_encode = pl.pallas_call(
    _encode_body,
    out_shape=[jax.ShapeDtypeStruct((_N, _H), jnp.float32),
               jax.ShapeDtypeStruct((_N, _H), jnp.float32)],
)


def _update_body(z_ref, u_ref, wmd_ref, bm_ref, wu_ref, bu_ref,
                 wt_ref, bt_ref, wnn_ref, bnn_ref,
                 wup_ref, bup_ref,
                 nh_ref, ns_ref, nne_ref, stop_ref):
    z = z_ref[...]
    zq = z.astype(jnp.bfloat16)
    u_raw = u_ref[...]
    b = jnp.dot(zq, wmd_ref[...], preferred_element_type=jnp.float32)
    u = jnp.where(jnp.isfinite(u_raw), u_raw + b + bm_ref[...], 0.0)
    nh = jnp.dot(jnp.concatenate([z, u], axis=1), wu_ref[...],
                 preferred_element_type=jnp.float32) + bu_ref[...]
    nh_ref[...] = nh
    loc = jnp.dot(nh, wt_ref[...], preferred_element_type=jnp.float32)
    mloc = jnp.dot(jnp.mean(nh, axis=0, keepdims=True), wt_ref[...],
                   preferred_element_type=jnp.float32)
    stop_ref[...] = jax.nn.sigmoid(
        jnp.maximum(jnp.max(loc, axis=0, keepdims=True), mloc) + bt_ref[...])
    nne = jnp.dot(jnp.concatenate([nh, z], axis=1), wnn_ref[...],
                  preferred_element_type=jnp.float32) + bnn_ref[...]
    nne_ref[...] = nne
    ns_ref[...] = jnp.dot(jnp.concatenate([nh, z, nne], axis=1), wup_ref[...],
                          preferred_element_type=jnp.float32) + bup_ref[...]


_update = pl.pallas_call(
    _update_body,
    out_shape=[jax.ShapeDtypeStruct((_N, _H), jnp.float32),
               jax.ShapeDtypeStruct((_N, 1), jnp.float32),
               jax.ShapeDtypeStruct((_N, 1), jnp.float32),
               jax.ShapeDtypeStruct((1, 1), jnp.float32)],
)

_SIG = list(range(0, _H, 2)) + list(range(1, _H, 2))


def kernel(states, priority, edges_mat, edge_index,
           W_enc, b_enc, W_M, b_M, W_U, b_U,
           W_nn, b_nn, W_up, b_up, W_t, b_t):
    src = edge_index[0]
    dst = edge_index[1]
    emat128 = edges_mat.astype(jnp.bfloat16).astype(
        jnp.float32).reshape(_N * _N // 128, 128)
    src_l, dst_l, feat_l, cnt = _prep(src, dst, emat128)

    bf = jnp.bfloat16
    be = b_enc[None]
    wms = W_M[:32].astype(bf)
    wmd = W_M[32:64].astype(bf)
    wf = W_M[64].astype(bf).astype(jnp.float32)
    bm = b_M[None]
    bu = b_U[None]
    bnn = b_nn[None]
    bup = b_up[None]
    bt = b_t[None]

    hidden = jnp.zeros((_N, _H), jnp.float32)
    x = states[0][:, None]
    prio = priority[:, None]
    pred_all = [x]
    pred_stop = [jnp.zeros((1, 1), jnp.float32)]
    pred_next = []
    for _ in range(_T - 1):
        z, a_f = _encode(x, hidden, prio, W_enc, be, wms)
        a0 = a_f[:, :16].reshape(_N * 16)
        a1 = a_f[:, 16:].reshape(_N * 16)
        uflat = _msgmax(a0, a1, src_l, dst_l, feat_l, cnt, wf)
        u_raw = jnp.concatenate(
            [uflat[:_N * 16].reshape(_N, 16),
             uflat[_N * 16:].reshape(_N, 16)], axis=1)
        nh, ns, nne, stop = _update(z, u_raw, wmd, bm, W_U, bu,
                                    W_t, bt, W_nn, bnn, W_up, bup)
        hidden = nh
        x = ns
        pred_all.append(ns)
        pred_stop.append(stop)
        pred_next.append(nne)

    preds = jnp.stack(pred_all, axis=1).reshape(_T, _N)
    preds_stop = jnp.stack(pred_stop, axis=1)
    preds_nextnode = jnp.stack(pred_next, axis=1)
    return (preds, preds_stop, preds_nextnode)
